# Initial kernel scaffold; baseline (speedup 1.0000x reference)
import functools

import jax
import jax.numpy as jnp
from jax import lax
from jax.experimental import pallas as pl
from jax.experimental.pallas import tpu as pltpu
from jax.experimental.pallas import tpu_sc as plsc

B = 16
S = 512
NLM = 5000
NTILES = 32
LANES = 16
PAIRS = (B * NLM) // NTILES
PAD_PAIRS = 2560
NVEC = PAD_PAIRS // LANES
ELEMS = PAIRS * 2
TOTAL = B * NLM * 2
PLANE = S * S


def _sc_body(flow_hbm, lmf_hbm, lms_hbm, gate_hbm, out_hbm,
             lmf_v, lms_v, gate_v, idx_v, pts_v, row_v, sem):
    cid = lax.axis_index("c")
    sid = lax.axis_index("s")
    wid = cid * 16 + sid
    batch = wid // 2
    base = wid * ELEMS
    plane0 = batch * (2 * PLANE)

    pltpu.sync_copy(lmf_hbm.at[pl.ds(base, ELEMS)], lmf_v)
    pltpu.sync_copy(lms_hbm.at[pl.ds(base, ELEMS)], lms_v)
    pltpu.sync_copy(gate_hbm.at[pl.ds(base, ELEMS)], gate_v)

    lanes = lax.iota(jnp.int32, 16)

    def idx_body(v, _):
        p = v * LANES + lanes
        pc = jnp.minimum(p, PAIRS - 1)
        x = plsc.load_gather(lmf_v, [2 * pc])
        y = plsc.load_gather(lmf_v, [2 * pc + 1])
        idx0 = plane0 + y * S + x
        idx_v[pl.ds(v * LANES, LANES)] = idx0
        idx_v[pl.ds(PAD_PAIRS + v * LANES, LANES)] = idx0 + PLANE
        return 0

    lax.fori_loop(0, NVEC, idx_body, 0)

    pltpu.async_copy(flow_hbm.at[idx_v], pts_v, sem).wait()

    inv_half_s = jnp.float32(2.0 / S)
    scale = jnp.float32(1.0 / TOTAL)

    def acc_body(v, acc):
        p = v * LANES + lanes
        pc = jnp.minimum(p, PAIRS - 1)
        g0 = plsc.load_gather(gate_v, [2 * pc])
        g1 = plsc.load_gather(gate_v, [2 * pc + 1])
        s0 = plsc.load_gather(lms_v, [2 * pc])
        s1 = plsc.load_gather(lms_v, [2 * pc + 1])
        gt0 = s0.astype(jnp.float32) * inv_half_s - 1.0
        gt1 = s1.astype(jnp.float32) * inv_half_s - 1.0
        pt0 = pts_v[pl.ds(v * LANES, LANES)]
        pt1 = pts_v[pl.ds(PAD_PAIRS + v * LANES, LANES)]
        d0 = pt0 * g0 - gt0 * g0
        d1 = pt1 * g1 - gt1 * g1
        sq = d0 * d0 + d1 * d1
        valid = p < PAIRS
        return acc + jnp.where(valid, sq, jnp.float32(0.0))

    acc = lax.fori_loop(0, NVEC, acc_body, jnp.zeros((16,), jnp.float32))
    row_v[...] = acc * scale
    pltpu.sync_copy(row_v, out_hbm.at[wid])


@jax.jit
def _landmark_loss(flow_flat, lmf_flat, lms_flat, gate_flat):
    mesh = plsc.VectorSubcoreMesh(core_axis_name="c", subcore_axis_name="s")
    run = functools.partial(
        pl.kernel,
        out_type=jax.ShapeDtypeStruct((NTILES, 16), jnp.float32),
        mesh=mesh,
        scratch_types=[
            pltpu.VMEM((ELEMS,), jnp.int32),
            pltpu.VMEM((ELEMS,), jnp.int32),
            pltpu.VMEM((ELEMS,), jnp.float32),
            pltpu.VMEM((2 * PAD_PAIRS,), jnp.int32),
            pltpu.VMEM((2 * PAD_PAIRS,), jnp.float32),
            pltpu.VMEM((16,), jnp.float32),
            pltpu.SemaphoreType.DMA,
        ],
    )(_sc_body)
    partials = run(flow_flat, lmf_flat, lms_flat, gate_flat)
    return jnp.sum(partials)


def kernel(flow, lm_S, lm_F, gate):
    return _landmark_loss(
        flow.reshape(-1),
        lm_F.reshape(-1),
        lm_S.reshape(-1),
        gate.reshape(-1),
    )


# speedup vs baseline: 1.1417x; 1.1417x over previous
import functools

import jax
import jax.numpy as jnp
from jax import lax
from jax.experimental import pallas as pl
from jax.experimental.pallas import tpu as pltpu
from jax.experimental.pallas import tpu_sc as plsc

B = 16
S = 512
NLM = 5000
NTILES = 32
LANES = 16
PAIRS = (B * NLM) // NTILES
PAD_PAIRS = 2560
NVEC = PAD_PAIRS // LANES
ELEMS = PAIRS * 2
TOTAL = B * NLM * 2
PLANE = S * S


def _sc_body(flow_hbm, lmf_hbm, lms_hbm, gate_hbm, out_hbm,
             lmf_v, lms_v, gate_v, idx_v, pts_v, row_v, sem):
    cid = lax.axis_index("c")
    sid = lax.axis_index("s")
    wid = cid * 16 + sid
    batch = wid // 2
    base = wid * ELEMS
    plane0 = batch * (2 * PLANE)

    pltpu.sync_copy(lmf_hbm.at[pl.ds(base, ELEMS)], lmf_v)
    pltpu.sync_copy(lms_hbm.at[pl.ds(base, ELEMS)], lms_v)
    pltpu.sync_copy(gate_hbm.at[pl.ds(base, ELEMS)], gate_v)

    lanes = lax.iota(jnp.int32, 16)

    def idx_body(v, _):
        p = v * LANES + lanes
        pc = jnp.minimum(p, PAIRS - 1)
        x = plsc.load_gather(lmf_v, [2 * pc])
        y = plsc.load_gather(lmf_v, [2 * pc + 1])
        idx0 = plane0 + y * S + x
        idx_v[pl.ds(v * LANES, LANES)] = idx0
        idx_v[pl.ds(PAD_PAIRS + v * LANES, LANES)] = idx0 + PLANE
        return 0

    lax.fori_loop(0, NVEC, idx_body, 0)

    pltpu.async_copy(flow_hbm.at[idx_v], pts_v, sem).wait()

    inv_half_s = jnp.float32(2.0 / S)
    scale = jnp.float32(1.0 / TOTAL)

    def acc_body(v, acc):
        p = v * LANES + lanes
        pc = jnp.minimum(p, PAIRS - 1)
        g0 = plsc.load_gather(gate_v, [2 * pc])
        g1 = plsc.load_gather(gate_v, [2 * pc + 1])
        s0 = plsc.load_gather(lms_v, [2 * pc])
        s1 = plsc.load_gather(lms_v, [2 * pc + 1])
        gt0 = s0.astype(jnp.float32) * inv_half_s - 1.0
        gt1 = s1.astype(jnp.float32) * inv_half_s - 1.0
        pt0 = pts_v[pl.ds(v * LANES, LANES)]
        pt1 = pts_v[pl.ds(PAD_PAIRS + v * LANES, LANES)]
        d0 = pt0 * g0 - gt0 * g0
        d1 = pt1 * g1 - gt1 * g1
        sq = d0 * d0 + d1 * d1
        valid = p < PAIRS
        return acc + jnp.where(valid, sq, jnp.float32(0.0))

    acc = lax.fori_loop(0, NVEC, acc_body, jnp.zeros((16,), jnp.float32))
    row_v[...] = acc * scale
    pltpu.sync_copy(row_v, out_hbm.at[wid])


@jax.jit
def _landmark_loss(flow_flat, lmf_flat, lms_flat, gate_flat):
    mesh = plsc.VectorSubcoreMesh(core_axis_name="c", subcore_axis_name="s")
    run = functools.partial(
        pl.kernel,
        out_type=jax.ShapeDtypeStruct((NTILES, 16), jnp.float32),
        mesh=mesh,
        scratch_types=[
            pltpu.VMEM((ELEMS,), jnp.int32),
            pltpu.VMEM((ELEMS,), jnp.int32),
            pltpu.VMEM((ELEMS,), jnp.float32),
            pltpu.VMEM((2 * PAD_PAIRS,), jnp.int32),
            pltpu.VMEM((2 * PAD_PAIRS,), jnp.float32),
            pltpu.VMEM((16,), jnp.float32),
            pltpu.SemaphoreType.DMA,
        ],
        compiler_params=pltpu.CompilerParams(needs_layout_passes=False),
    )(_sc_body)
    partials = run(flow_flat, lmf_flat, lms_flat, gate_flat)
    return jnp.sum(partials)


def kernel(flow, lm_S, lm_F, gate):
    return _landmark_loss(
        flow.reshape(-1),
        lm_F.reshape(-1),
        lm_S.reshape(-1),
        gate.reshape(-1),
    )


# speedup vs baseline: 1.2657x; 1.1086x over previous
import functools

import jax
import jax.numpy as jnp
from jax import lax
from jax.experimental import pallas as pl
from jax.experimental.pallas import tpu as pltpu
from jax.experimental.pallas import tpu_sc as plsc

B = 16
S = 512
NLM = 5000
NTILES = 32
LANES = 16
PAIRS = (B * NLM) // NTILES
PAD_PAIRS = 2560
NVEC = PAD_PAIRS // LANES
ELEMS = PAIRS * 2
TOTAL = B * NLM * 2
PLANE = S * S


def _sc_body(flow_hbm, lmf_hbm, lms_hbm, gate_hbm, out_hbm,
             lmf_v, lms_v, gate_v, idx_v, pts_v, row_v, sem):
    cid = lax.axis_index("c")
    sid = lax.axis_index("s")
    wid = cid * 16 + sid
    batch = wid // 2
    base = wid * ELEMS
    plane0 = batch * (2 * PLANE)

    pltpu.sync_copy(lmf_hbm.at[pl.ds(base, ELEMS)], lmf_v)
    pltpu.sync_copy(lms_hbm.at[pl.ds(base, ELEMS)], lms_v)
    pltpu.sync_copy(gate_hbm.at[pl.ds(base, ELEMS)], gate_v)

    lanes = lax.iota(jnp.int32, 16)

    def idx_body(v, _):
        p = v * LANES + lanes
        pc = jnp.minimum(p, PAIRS - 1)
        x = plsc.load_gather(lmf_v, [2 * pc])
        y = plsc.load_gather(lmf_v, [2 * pc + 1])
        within = (((y >> 3) * 4 + (x >> 7)) << 10) + ((y & 7) << 7) + (x & 127)
        idx0 = plane0 + within
        idx_v[pl.ds(v * LANES, LANES)] = idx0
        idx_v[pl.ds(PAD_PAIRS + v * LANES, LANES)] = idx0 + PLANE
        return 0

    lax.fori_loop(0, NVEC, idx_body, 0)

    pltpu.async_copy(flow_hbm.at[idx_v], pts_v, sem).wait()

    inv_half_s = jnp.float32(2.0 / S)
    scale = jnp.float32(1.0 / TOTAL)

    def acc_body(v, acc):
        p = v * LANES + lanes
        pc = jnp.minimum(p, PAIRS - 1)
        g0 = plsc.load_gather(gate_v, [2 * pc])
        g1 = plsc.load_gather(gate_v, [2 * pc + 1])
        s0 = plsc.load_gather(lms_v, [2 * pc])
        s1 = plsc.load_gather(lms_v, [2 * pc + 1])
        gt0 = s0.astype(jnp.float32) * inv_half_s - 1.0
        gt1 = s1.astype(jnp.float32) * inv_half_s - 1.0
        pt0 = pts_v[pl.ds(v * LANES, LANES)]
        pt1 = pts_v[pl.ds(PAD_PAIRS + v * LANES, LANES)]
        d0 = pt0 * g0 - gt0 * g0
        d1 = pt1 * g1 - gt1 * g1
        sq = d0 * d0 + d1 * d1
        valid = p < PAIRS
        return acc + jnp.where(valid, sq, jnp.float32(0.0))

    acc = lax.fori_loop(0, NVEC, acc_body, jnp.zeros((16,), jnp.float32))
    row_v[...] = acc * scale
    pltpu.sync_copy(row_v, out_hbm.at[wid])


@jax.jit
def _landmark_loss(flow_flat, lmf_flat, lms_flat, gate_flat):
    mesh = plsc.VectorSubcoreMesh(core_axis_name="c", subcore_axis_name="s")
    run = functools.partial(
        pl.kernel,
        out_type=jax.ShapeDtypeStruct((NTILES, 16), jnp.float32),
        mesh=mesh,
        scratch_types=[
            pltpu.VMEM((ELEMS,), jnp.int32),
            pltpu.VMEM((ELEMS,), jnp.int32),
            pltpu.VMEM((ELEMS,), jnp.float32),
            pltpu.VMEM((2 * PAD_PAIRS,), jnp.int32),
            pltpu.VMEM((2 * PAD_PAIRS,), jnp.float32),
            pltpu.VMEM((16,), jnp.float32),
            pltpu.SemaphoreType.DMA,
        ],
        compiler_params=pltpu.CompilerParams(needs_layout_passes=False),
    )(_sc_body)
    partials = run(flow_flat, lmf_flat, lms_flat, gate_flat)
    return jnp.sum(partials)


def kernel(flow, lm_S, lm_F, gate):
    flow_t = (
        flow.reshape(B, 2, S // 8, 8, S // 128, 128)
        .transpose(0, 1, 2, 4, 3, 5)
        .reshape(-1)
    )
    return _landmark_loss(
        flow_t,
        lm_F.reshape(-1),
        lm_S.reshape(-1),
        gate.reshape(-1),
    )


# speedup vs baseline: 4.9149x; 3.8830x over previous
import functools

import jax
import jax.numpy as jnp
from jax import lax
from jax.experimental import pallas as pl
from jax.experimental.pallas import tpu as pltpu
from jax.experimental.pallas import tpu_sc as plsc

B = 16
S = 512
NLM = 5000
NPAIRS = B * NLM
NTILES = 32
LANES = 16
PAIRS = NPAIRS // NTILES
WIN = 2504
PAD = 2512
NVEC = PAD // LANES
TOTAL = NPAIRS * 2
PLANE = S * S


def _sc_body(flow_hbm, x_hbm, y_hbm, sx_hbm, sy_hbm, g0_hbm, g1_hbm, out_hbm,
             x_v, y_v, sx_v, sy_v, g0_v, g1_v, idx_v, pts_v, row_v, sem):
    cid = lax.axis_index("c")
    sid = lax.axis_index("s")
    wid = cid * 16 + sid
    batch = wid // 2
    half = wid % 2
    w0 = half * (NLM - WIN)
    lo = half * 4
    plane0 = batch * (2 * PLANE)

    for src, buf in zip((x_hbm, y_hbm, sx_hbm, sy_hbm, g0_hbm, g1_hbm),
                        (x_v, y_v, sx_v, sy_v, g0_v, g1_v)):
        pltpu.sync_copy(src.at[pl.ds(batch * NLM + w0, WIN)],
                        buf.at[pl.ds(0, WIN)])

    lanes = lax.iota(jnp.int32, 16)

    def idx_body(v, _):
        p = v * LANES + lanes
        x = x_v[pl.ds(v * LANES, LANES)]
        y = y_v[pl.ds(v * LANES, LANES)]
        within = (((y >> 3) * 4 + (x >> 7)) << 10) + ((y & 7) << 7) + (x & 127)
        idx0 = jnp.where(p < WIN, plane0 + within, 0)
        idx_v[pl.ds(v * LANES, LANES)] = idx0
        idx_v[pl.ds(PAD + v * LANES, LANES)] = idx0 + PLANE
        return 0

    lax.fori_loop(0, NVEC, idx_body, 0)

    pltpu.async_copy(flow_hbm.at[idx_v], pts_v, sem).wait()

    inv_half_s = jnp.float32(2.0 / S)
    scale = jnp.float32(1.0 / TOTAL)

    def acc_body(v, acc):
        p = v * LANES + lanes
        g0 = plsc.bitcast(g0_v[pl.ds(v * LANES, LANES)], jnp.float32)
        g1 = plsc.bitcast(g1_v[pl.ds(v * LANES, LANES)], jnp.float32)
        s0 = sx_v[pl.ds(v * LANES, LANES)]
        s1 = sy_v[pl.ds(v * LANES, LANES)]
        gt0 = s0.astype(jnp.float32) * inv_half_s - 1.0
        gt1 = s1.astype(jnp.float32) * inv_half_s - 1.0
        pt0 = pts_v[pl.ds(v * LANES, LANES)]
        pt1 = pts_v[pl.ds(PAD + v * LANES, LANES)]
        d0 = pt0 * g0 - gt0 * g0
        d1 = pt1 * g1 - gt1 * g1
        sq = d0 * d0 + d1 * d1
        valid = (p >= lo) & (p < lo + PAIRS)
        return acc + jnp.where(valid, sq, jnp.float32(0.0))

    acc = lax.fori_loop(0, NVEC, acc_body, jnp.zeros((16,), jnp.float32))
    row_v[...] = acc * scale
    pltpu.sync_copy(row_v, out_hbm.at[wid])


@jax.jit
def _landmark_loss(flow_flat, x_f, y_f, sx_f, sy_f, g0_f, g1_f):
    mesh = plsc.VectorSubcoreMesh(core_axis_name="c", subcore_axis_name="s")
    run = functools.partial(
        pl.kernel,
        out_type=jax.ShapeDtypeStruct((NTILES, 16), jnp.float32),
        mesh=mesh,
        scratch_types=[
            pltpu.VMEM((PAD,), jnp.int32),
            pltpu.VMEM((PAD,), jnp.int32),
            pltpu.VMEM((PAD,), jnp.int32),
            pltpu.VMEM((PAD,), jnp.int32),
            pltpu.VMEM((PAD,), jnp.int32),
            pltpu.VMEM((PAD,), jnp.int32),
            pltpu.VMEM((2 * PAD,), jnp.int32),
            pltpu.VMEM((2 * PAD,), jnp.float32),
            pltpu.VMEM((16,), jnp.float32),
            pltpu.SemaphoreType.DMA,
        ],
        compiler_params=pltpu.CompilerParams(needs_layout_passes=False),
    )(_sc_body)
    partials = run(flow_flat, x_f, y_f, sx_f, sy_f, g0_f, g1_f)
    return jnp.sum(partials)


def kernel(flow, lm_S, lm_F, gate):
    flow_t = (
        flow.reshape(B, 2, S // 8, 8, S // 128, 128)
        .transpose(0, 1, 2, 4, 3, 5)
        .reshape(-1)
    )
    gate_i = jax.lax.bitcast_convert_type(gate, jnp.int32)
    return _landmark_loss(
        flow_t,
        lm_F[:, :, 0].reshape(-1),
        lm_F[:, :, 1].reshape(-1),
        lm_S[:, :, 0].reshape(-1),
        lm_S[:, :, 1].reshape(-1),
        gate_i[:, :, 0].reshape(-1),
        gate_i[:, :, 1].reshape(-1),
    )
